# baseline (speedup 1.0000x reference)
import functools

import jax
import jax.numpy as jnp
from jax import lax
from jax.experimental import pallas as pl
from jax.experimental.pallas import tpu as pltpu
from jax.experimental.pallas import tpu_sc as plsc

_BATCH = 16384
_EMB = 32
_GATHER_ON = True


def _make_sc_gather_t(batch, emb_dim, vocab):
    info = plsc.get_sparse_core_info()
    nc, ns = info.num_cores, info.num_subcores
    half = emb_dim // nc
    ichunk = 2048
    nt = batch // 128
    mesh = plsc.VectorSubcoreMesh(core_axis_name="c", subcore_axis_name="s")

    @functools.partial(
        pl.kernel,
        out_type=jax.ShapeDtypeStruct((2 * emb_dim, nt, 128), jnp.float32),
        mesh=mesh,
        compiler_params=pltpu.CompilerParams(needs_layout_passes=False),
        scratch_types=[
            pltpu.VMEM((vocab,), jnp.float32),
            pltpu.VMEM((ichunk,), jnp.int32),
            pltpu.VMEM((ichunk,), jnp.int32),
            pltpu.VMEM((ichunk // 128, 128), jnp.float32),
            pltpu.VMEM((ichunk // 128, 128), jnp.float32),
        ],
    )
    def sc_gather(tT_hbm, ia_hbm, ib_hbm, rows_hbm,
                  row_v, ia_v, ib_v, ra_v, rb_v):
        c = lax.axis_index("c")
        s = lax.axis_index("s")
        d = c * half + s

        pltpu.sync_copy(tT_hbm.at[d], row_v)

        def gather_chunk(k, _):
            base = k * ichunk
            pltpu.sync_copy(ia_hbm.at[pl.ds(base, ichunk)], ia_v)
            pltpu.sync_copy(ib_hbm.at[pl.ds(base, ichunk)], ib_v)

            def gather_one(j, _):
                off = j * 16
                t, l = off // 128, off % 128
                ra_v[t, pl.ds(l, 16)] = plsc.load_gather(row_v, [ia_v[pl.ds(off, 16)]])
                rb_v[t, pl.ds(l, 16)] = plsc.load_gather(row_v, [ib_v[pl.ds(off, 16)]])
                return 0

            if _GATHER_ON:
                lax.fori_loop(0, ichunk // 16, gather_one, 0)
            t0 = pl.multiple_of(base // 128, ichunk // 128)
            pltpu.sync_copy(ra_v, rows_hbm.at[d, pl.ds(t0, ichunk // 128)])
            pltpu.sync_copy(rb_v, rows_hbm.at[emb_dim + d, pl.ds(t0, ichunk // 128)])
            return 0

        lax.fori_loop(0, batch // ichunk, gather_chunk, 0, unroll=False)

    return sc_gather


def _make_sc_xpose(batch, emb_dim):
    info = plsc.get_sparse_core_info()
    nw = info.num_cores * info.num_subcores
    nt = batch // 128
    per_w = batch // nw
    tpw = per_w // 128
    two_d = 2 * emb_dim
    mesh = plsc.VectorSubcoreMesh(core_axis_name="c", subcore_axis_name="s")

    @functools.partial(
        pl.kernel,
        out_type=jax.ShapeDtypeStruct((batch, 128), jnp.float32),
        mesh=mesh,
        compiler_params=pltpu.CompilerParams(use_tc_tiling_on_sc=False,
                                             needs_layout_passes=False),
        scratch_types=[
            pltpu.VMEM((two_d, tpw, 128), jnp.float32),
            pltpu.VMEM((per_w, two_d), jnp.float32),
        ],
    )
    def sc_xpose(rows_hbm, comb_hbm, xch_v, out_t):
        wid = lax.axis_index("s") * info.num_cores + lax.axis_index("c")
        b0 = wid * per_w
        pltpu.sync_copy(rows_hbm.at[:, pl.ds(wid * tpw, tpw)], xch_v)
        lane16 = lax.broadcasted_iota(jnp.int32, (16,), 0)

        def xpose_one(t, _):
            r = t % two_d
            bb = (t // two_d) * 16
            vals = xch_v[r, bb // 128, pl.ds(bb % 128, 16)]
            plsc.store_scatter(
                out_t, [bb + lane16, jnp.full((16,), r, jnp.int32)], vals)
            return 0

        lax.fori_loop(0, two_d * per_w // 16, xpose_one, 0)
        pltpu.sync_copy(out_t, comb_hbm.at[pl.ds(b0, per_w), pl.ds(0, two_d)])

    return sc_xpose


_sc_gather_t = _make_sc_gather_t(_BATCH, _EMB, 100000)
_sc_xpose = _make_sc_xpose(_BATCH, _EMB)


def _dot_t(a, w):
    return lax.dot_general(a, w, (((1,), (1,)), ((), ())),
                           preferred_element_type=jnp.float32)


def _mlp_body(comb, ft, w1, b1, w2, b2, w3, b3, w4, out):
    x = jnp.concatenate([comb[:, 0:64], ft[...]], axis=1)
    h = jnp.maximum(_dot_t(x, w1[...]) + b1[...], 0.0)
    h = jnp.maximum(_dot_t(h, w2[...]) + b2[...], 0.0)
    h = jnp.maximum(_dot_t(h, w3[...]) + b3[...], 0.0)
    h = jnp.concatenate([h, jnp.ones((h.shape[0], 1), jnp.float32)], axis=1)
    out[...] = jax.nn.sigmoid(_dot_t(h, w4[...]))


def _mlp(comb, ft, W1, b1, W2, b2, W3, b3, W4e, blk=2048):
    batch = comb.shape[0]
    grid = (batch // blk,)
    full = lambda a: pl.BlockSpec(a.shape, lambda i: (0,) * a.ndim)
    row = lambda a: pl.BlockSpec((blk, a.shape[1]), lambda i: (i, 0))
    return pl.pallas_call(
        _mlp_body,
        grid=grid,
        in_specs=[
            row(comb), row(ft),
            full(W1), full(b1), full(W2), full(b2),
            full(W3), full(b3), full(W4e),
        ],
        out_specs=pl.BlockSpec((blk, 1), lambda i: (i, 0)),
        out_shape=jax.ShapeDtypeStruct((batch, 1), jnp.float32),
    )(comb, ft, W1, b1, W2, b2, W3, b3, W4e)


def kernel(airport_a, airport_b, features, table,
           W1, b1, W2, b2, W3, b3, W4, b4):
    ia = airport_a.astype(jnp.int32)
    ib = airport_b.astype(jnp.int32)
    rows = _sc_gather_t(table.T, ia, ib)
    comb = _sc_xpose(rows)
    w4e = jnp.concatenate([W4, b4.reshape(1, 1)], axis=1)
    out = _mlp(comb, features,
               W1, b1.reshape(1, -1), W2, b2.reshape(1, -1),
               W3, b3.reshape(1, -1), w4e)
    return out[:, 0]


# speedup vs baseline: 1.0310x; 1.0310x over previous
import functools

import jax
import jax.numpy as jnp
from jax import lax
from jax.experimental import pallas as pl
from jax.experimental.pallas import tpu as pltpu
from jax.experimental.pallas import tpu_sc as plsc

_BATCH = 16384
_EMB = 32


def _make_sc_gather_t(batch, emb_dim, vocab):
    info = plsc.get_sparse_core_info()
    nc, ns = info.num_cores, info.num_subcores
    half = emb_dim // nc
    ichunk = 4096
    nt = batch // 128
    mesh = plsc.VectorSubcoreMesh(core_axis_name="c", subcore_axis_name="s")

    @functools.partial(
        pl.kernel,
        out_type=jax.ShapeDtypeStruct((2 * emb_dim, nt, 128), jnp.float32),
        mesh=mesh,
        compiler_params=pltpu.CompilerParams(needs_layout_passes=False),
        scratch_types=[
            pltpu.VMEM((vocab,), jnp.float32),
            pltpu.VMEM((ichunk,), jnp.int32),
            pltpu.VMEM((ichunk,), jnp.int32),
            pltpu.VMEM((ichunk // 128, 128), jnp.float32),
            pltpu.VMEM((ichunk // 128, 128), jnp.float32),
        ],
    )
    def sc_gather(tT_hbm, ia_hbm, ib_hbm, rows_hbm,
                  row_v, ia_v, ib_v, ra_v, rb_v):
        c = lax.axis_index("c")
        s = lax.axis_index("s")
        d = c * half + s

        pltpu.sync_copy(tT_hbm.at[d], row_v)

        def gather_chunk(k, _):
            base = k * ichunk
            pltpu.sync_copy(ia_hbm.at[pl.ds(base, ichunk)], ia_v)
            pltpu.sync_copy(ib_hbm.at[pl.ds(base, ichunk)], ib_v)

            def gather_one(j, _):
                off = j * 16
                t, l = off // 128, off % 128
                ra_v[t, pl.ds(l, 16)] = plsc.load_gather(row_v, [ia_v[pl.ds(off, 16)]])
                rb_v[t, pl.ds(l, 16)] = plsc.load_gather(row_v, [ib_v[pl.ds(off, 16)]])
                return 0

            lax.fori_loop(0, ichunk // 16, gather_one, 0, unroll=8)
            t0 = pl.multiple_of(base // 128, ichunk // 128)
            pltpu.sync_copy(ra_v, rows_hbm.at[d, pl.ds(t0, ichunk // 128)])
            pltpu.sync_copy(rb_v, rows_hbm.at[emb_dim + d, pl.ds(t0, ichunk // 128)])
            return 0

        lax.fori_loop(0, batch // ichunk, gather_chunk, 0, unroll=False)

    return sc_gather


def _make_sc_xpose(batch, emb_dim):
    info = plsc.get_sparse_core_info()
    nw = info.num_cores * info.num_subcores
    nt = batch // 128
    per_w = batch // nw
    tpw = per_w // 128
    two_d = 2 * emb_dim
    mesh = plsc.VectorSubcoreMesh(core_axis_name="c", subcore_axis_name="s")

    @functools.partial(
        pl.kernel,
        out_type=jax.ShapeDtypeStruct((batch, 128), jnp.float32),
        mesh=mesh,
        compiler_params=pltpu.CompilerParams(use_tc_tiling_on_sc=False,
                                             needs_layout_passes=False),
        scratch_types=[
            pltpu.VMEM((two_d, tpw, 128), jnp.float32),
            pltpu.VMEM((per_w, two_d), jnp.float32),
        ],
    )
    def sc_xpose(rows_hbm, comb_hbm, xch_v, out_t):
        wid = lax.axis_index("s") * info.num_cores + lax.axis_index("c")
        b0 = wid * per_w
        pltpu.sync_copy(rows_hbm.at[:, pl.ds(wid * tpw, tpw)], xch_v)
        lane16 = lax.broadcasted_iota(jnp.int32, (16,), 0)

        def xpose_one(t, _):
            r = t % two_d
            bb = (t // two_d) * 16
            vals = xch_v[r, bb // 128, pl.ds(bb % 128, 16)]
            plsc.store_scatter(
                out_t, [bb + lane16, jnp.full((16,), r, jnp.int32)], vals)
            return 0

        lax.fori_loop(0, two_d * per_w // 16, xpose_one, 0, unroll=8)
        pltpu.sync_copy(out_t, comb_hbm.at[pl.ds(b0, per_w), pl.ds(0, two_d)])

    return sc_xpose


_sc_gather_t = _make_sc_gather_t(_BATCH, _EMB, 100000)
_sc_xpose = _make_sc_xpose(_BATCH, _EMB)


def _dot_t(a, w):
    return lax.dot_general(a, w, (((1,), (1,)), ((), ())),
                           preferred_element_type=jnp.float32)


def _mlp_body(comb, ft, w1, b1, w2, b2, w3, b3, w4, out):
    x = jnp.concatenate([comb[:, 0:64], ft[...]], axis=1)
    h = jnp.maximum(_dot_t(x, w1[...]) + b1[...], 0.0)
    h = jnp.maximum(_dot_t(h, w2[...]) + b2[...], 0.0)
    h = jnp.maximum(_dot_t(h, w3[...]) + b3[...], 0.0)
    h = jnp.concatenate([h, jnp.ones((h.shape[0], 1), jnp.float32)], axis=1)
    out[...] = jax.nn.sigmoid(_dot_t(w4[...], h))


def _mlp(comb, ft, W1, b1, W2, b2, W3, b3, W4e, blk=2048):
    batch = comb.shape[0]
    grid = (batch // blk,)
    full = lambda a: pl.BlockSpec(a.shape, lambda i: (0,) * a.ndim)
    row = lambda a: pl.BlockSpec((blk, a.shape[1]), lambda i: (i, 0))
    return pl.pallas_call(
        _mlp_body,
        grid=grid,
        in_specs=[
            row(comb), row(ft),
            full(W1), full(b1), full(W2), full(b2),
            full(W3), full(b3), full(W4e),
        ],
        out_specs=pl.BlockSpec((1, blk), lambda i: (0, i)),
        out_shape=jax.ShapeDtypeStruct((1, batch), jnp.float32),
    )(comb, ft, W1, b1, W2, b2, W3, b3, W4e)


def kernel(airport_a, airport_b, features, table,
           W1, b1, W2, b2, W3, b3, W4, b4):
    ia = airport_a.astype(jnp.int32)
    ib = airport_b.astype(jnp.int32)
    rows = _sc_gather_t(table.T, ia, ib)
    comb = _sc_xpose(rows)
    w4e = jnp.concatenate([W4, b4.reshape(1, 1)], axis=1)
    out = _mlp(comb, features,
               W1, b1.reshape(1, -1), W2, b2.reshape(1, -1),
               W3, b3.reshape(1, -1), w4e)
    return out.reshape(-1)


# speedup vs baseline: 1.2936x; 1.2548x over previous
import functools

import jax
import jax.numpy as jnp
from jax import lax
from jax.experimental import pallas as pl
from jax.experimental.pallas import tpu as pltpu
from jax.experimental.pallas import tpu_sc as plsc

_BATCH = 16384
_EMB = 32


def _make_sc_gather_t(batch, emb_dim, vocab):
    info = plsc.get_sparse_core_info()
    nc, ns = info.num_cores, info.num_subcores
    half = emb_dim // nc
    ichunk = 4096
    nt = batch // 128
    mesh = plsc.VectorSubcoreMesh(core_axis_name="c", subcore_axis_name="s")

    @functools.partial(
        pl.kernel,
        out_type=jax.ShapeDtypeStruct((2 * emb_dim, nt, 128), jnp.float32),
        mesh=mesh,
        compiler_params=pltpu.CompilerParams(needs_layout_passes=False),
        scratch_types=[
            pltpu.VMEM((vocab,), jnp.float32),
            pltpu.VMEM((ichunk,), jnp.int32),
            pltpu.VMEM((ichunk,), jnp.int32),
            pltpu.VMEM((ichunk // 128, 128), jnp.float32),
            pltpu.VMEM((ichunk // 128, 128), jnp.float32),
        ],
    )
    def sc_gather(tT_hbm, ia_hbm, ib_hbm, rows_hbm,
                  row_v, ia_v, ib_v, ra_v, rb_v):
        c = lax.axis_index("c")
        s = lax.axis_index("s")
        d = c * half + s

        pltpu.sync_copy(tT_hbm.at[d], row_v)

        def gather_chunk(k, _):
            base = k * ichunk
            pltpu.sync_copy(ia_hbm.at[pl.ds(base, ichunk)], ia_v)
            pltpu.sync_copy(ib_hbm.at[pl.ds(base, ichunk)], ib_v)

            @plsc.parallel_loop(0, ichunk // 16, unroll=8)
            def gather_one(j):
                off = j * 16
                t, l = off // 128, off % 128
                ra_v[t, pl.ds(l, 16)] = plsc.load_gather(row_v, [ia_v[pl.ds(off, 16)]])
                rb_v[t, pl.ds(l, 16)] = plsc.load_gather(row_v, [ib_v[pl.ds(off, 16)]])
            t0 = pl.multiple_of(base // 128, ichunk // 128)
            pltpu.sync_copy(ra_v, rows_hbm.at[d, pl.ds(t0, ichunk // 128)])
            pltpu.sync_copy(rb_v, rows_hbm.at[emb_dim + d, pl.ds(t0, ichunk // 128)])
            return 0

        lax.fori_loop(0, batch // ichunk, gather_chunk, 0, unroll=False)

    return sc_gather


def _make_sc_xpose(batch, emb_dim):
    info = plsc.get_sparse_core_info()
    nw = info.num_cores * info.num_subcores
    nt = batch // 128
    per_w = batch // nw
    tpw = per_w // 128
    two_d = 2 * emb_dim
    mesh = plsc.VectorSubcoreMesh(core_axis_name="c", subcore_axis_name="s")

    @functools.partial(
        pl.kernel,
        out_type=jax.ShapeDtypeStruct((batch, 128), jnp.float32),
        mesh=mesh,
        compiler_params=pltpu.CompilerParams(use_tc_tiling_on_sc=False,
                                             needs_layout_passes=False),
        scratch_types=[
            pltpu.VMEM((two_d, tpw, 128), jnp.float32),
            pltpu.VMEM((per_w, two_d), jnp.float32),
        ],
    )
    def sc_xpose(rows_hbm, comb_hbm, xch_v, out_t):
        wid = lax.axis_index("s") * info.num_cores + lax.axis_index("c")
        b0 = wid * per_w
        pltpu.sync_copy(rows_hbm.at[:, pl.ds(wid * tpw, tpw)], xch_v)
        lane16 = lax.broadcasted_iota(jnp.int32, (16,), 0)

        @plsc.parallel_loop(0, two_d * per_w // 16, unroll=8)
        def xpose_one(t):
            r = t % two_d
            bb = (t // two_d) * 16
            vals = xch_v[r, bb // 128, pl.ds(bb % 128, 16)]
            plsc.store_scatter(
                out_t, [bb + lane16, jnp.full((16,), r, jnp.int32)], vals)
        pltpu.sync_copy(out_t, comb_hbm.at[pl.ds(b0, per_w), pl.ds(0, two_d)])

    return sc_xpose


_sc_gather_t = _make_sc_gather_t(_BATCH, _EMB, 100000)
_sc_xpose = _make_sc_xpose(_BATCH, _EMB)


def _dot_t(a, w):
    return lax.dot_general(a, w, (((1,), (1,)), ((), ())),
                           preferred_element_type=jnp.float32)


def _mlp_body(comb, ft, w1, b1, w2, b2, w3, b3, w4, out):
    x = jnp.concatenate([comb[:, 0:64], ft[...]], axis=1)
    h = jnp.maximum(_dot_t(x, w1[...]) + b1[...], 0.0)
    h = jnp.maximum(_dot_t(h, w2[...]) + b2[...], 0.0)
    h = jnp.maximum(_dot_t(h, w3[...]) + b3[...], 0.0)
    h = jnp.concatenate([h, jnp.ones((h.shape[0], 1), jnp.float32)], axis=1)
    out[...] = jax.nn.sigmoid(_dot_t(w4[...], h))


def _mlp(comb, ft, W1, b1, W2, b2, W3, b3, W4e, blk=2048):
    batch = comb.shape[0]
    grid = (batch // blk,)
    full = lambda a: pl.BlockSpec(a.shape, lambda i: (0,) * a.ndim)
    row = lambda a: pl.BlockSpec((blk, a.shape[1]), lambda i: (i, 0))
    return pl.pallas_call(
        _mlp_body,
        grid=grid,
        in_specs=[
            row(comb), row(ft),
            full(W1), full(b1), full(W2), full(b2),
            full(W3), full(b3), full(W4e),
        ],
        out_specs=pl.BlockSpec((1, blk), lambda i: (0, i)),
        out_shape=jax.ShapeDtypeStruct((1, batch), jnp.float32),
    )(comb, ft, W1, b1, W2, b2, W3, b3, W4e)


def kernel(airport_a, airport_b, features, table,
           W1, b1, W2, b2, W3, b3, W4, b4):
    ia = airport_a.astype(jnp.int32)
    ib = airport_b.astype(jnp.int32)
    rows = _sc_gather_t(table.T, ia, ib)
    comb = _sc_xpose(rows)
    w4e = jnp.concatenate([W4, b4.reshape(1, 1)], axis=1)
    out = _mlp(comb, features,
               W1, b1.reshape(1, -1), W2, b2.reshape(1, -1),
               W3, b3.reshape(1, -1), w4e)
    return out.reshape(-1)
